# baseline (speedup 1.0000x reference)
import functools

import jax
import jax.numpy as jnp
from jax import lax
from jax.experimental import pallas as pl
from jax.experimental.pallas import tpu as pltpu
from jax.experimental.pallas import tpu_sc as plsc

N = 10000
E = 320000
D = 128
NEG = 0.01

NC = 2
NS = 16
K = 128
CPT = 80
EPAD = NC * NS * CPT * K
NPAD = 10240
RPT = NPAD // NS
_mesh = plsc.VectorSubcoreMesh(core_axis_name="c", subcore_axis_name="s")


@functools.partial(
    pl.kernel,
    out_type=jax.ShapeDtypeStruct((NC, NPAD), jnp.float32),
    mesh=_mesh,
    scratch_types=[
        pltpu.VMEM_SHARED((NPAD,), jnp.float32),
        pltpu.VMEM((CPT, K), jnp.int32),
        pltpu.VMEM((K,), jnp.float32),
        pltpu.VMEM((RPT,), jnp.float32),
    ],
)
def _deg_call(dst_hbm, out_hbm, dsh, didx, ones, zbuf):
    c = lax.axis_index("c")
    s = lax.axis_index("s")
    wid = c * NS + s

    def fill_ones(i, _):
        ones[pl.ds(i * 16, 16)] = jnp.full((16,), 1.0, jnp.float32)
        return 0

    lax.fori_loop(0, K // 16, fill_ones, 0)

    def fill_z(i, _):
        zbuf[pl.ds(i * 16, 16)] = jnp.zeros((16,), jnp.float32)
        return 0

    lax.fori_loop(0, RPT // 16, fill_z, 0)

    base = s * RPT
    pltpu.sync_copy(zbuf, dsh.at[pl.ds(base, RPT)])
    plsc.subcore_barrier()

    pltpu.sync_copy(dst_hbm.at[pl.ds(wid * CPT, CPT)], didx)

    def chunk(j, _):
        pltpu.sync_copy(ones, dsh.at[didx.at[j]], add=True)
        return 0

    lax.fori_loop(0, CPT, chunk, 0)
    plsc.subcore_barrier()
    pltpu.sync_copy(dsh.at[pl.ds(base, RPT)], out_hbm.at[c, pl.ds(base, RPT)])


K2 = 64
CPT2 = 160
STG = 4
CLOC = CPT2 // STG


@functools.partial(
    pl.kernel,
    out_type=jax.ShapeDtypeStruct((NC, NPAD, D), jnp.float32),
    mesh=_mesh,
    scratch_types=[
        pltpu.VMEM_SHARED((NPAD, D), jnp.float32),
        pltpu.VMEM((CLOC, K2), jnp.int32),
        pltpu.VMEM((CLOC, K2), jnp.int32),
        pltpu.VMEM((K2, D), jnp.float32),
        pltpu.VMEM((K2, D), jnp.float32),
        pltpu.VMEM((K2, D), jnp.float32),
        pltpu.VMEM((K2, D), jnp.float32),
        pltpu.SemaphoreType.DMA,
        pltpu.SemaphoreType.DMA,
        pltpu.SemaphoreType.DMA,
        pltpu.SemaphoreType.DMA,
    ],
)
def _scat_call(y_hbm, src_hbm, dst_hbm, out_hbm, zsh,
               sidx, didx, g0, g1, g2, g3, sem0, sem1, sem2, sem3):
    c = lax.axis_index("c")
    s = lax.axis_index("s")
    wid = c * NS + s

    def fill_z(i, _):
        def fill_col(j, _):
            g0[i, pl.ds(j * 16, 16)] = jnp.zeros((16,), jnp.float32)
            return 0

        return lax.fori_loop(0, D // 16, fill_col, 0)

    lax.fori_loop(0, K2, fill_z, 0)

    base = s * RPT
    for t in range(RPT // K2):
        pltpu.sync_copy(g0, zsh.at[pl.ds(base + t * K2, K2)])
    plsc.subcore_barrier()

    bufs = ((g0, sem0), (g1, sem1), (g2, sem2), (g3, sem3))

    def start(j, b):
        g, sem = bufs[b]
        pltpu.async_copy(y_hbm.at[sidx.at[j]], g, sem)

    def drain(j, b):
        g, sem = bufs[b]
        pltpu.make_async_copy(y_hbm.at[sidx.at[j]], g, sem).wait()
        pltpu.sync_copy(g, zsh.at[didx.at[j]], add=True)

    for st in range(STG):
        row0 = wid * CPT2 + st * CLOC
        pltpu.sync_copy(src_hbm.at[pl.ds(row0, CLOC)], sidx)
        pltpu.sync_copy(dst_hbm.at[pl.ds(row0, CLOC)], didx)

        for b in range(3):
            start(b, b)

        def quad(i, _):
            a = 4 * i
            start(a + 3, 3)
            for b in range(3):
                drain(a + b, b)
                start(a + b + 4, b)
            drain(a + 3, 3)
            return 0

        lax.fori_loop(0, CLOC // 4 - 1, quad, 0)
        a = CLOC - 4
        start(CLOC - 1, 3)
        for b in range(4):
            drain(a + b, b)

    plsc.subcore_barrier()
    pltpu.sync_copy(zsh.at[pl.ds(base, RPT)], out_hbm.at[c, pl.ds(base, RPT)])


_R = 2000


def _tc1(x, w1, dega, degb):
    def body(x_ref, w_ref, da_ref, db_ref, y_ref, dinv_ref):
        d = lax.rsqrt(da_ref[...] + db_ref[...] + 1.0)
        y_ref[...] = jnp.dot(x_ref[...], w_ref[...], preferred_element_type=jnp.float32) * d
        dinv_ref[...] = d

    return pl.pallas_call(
        body,
        grid=(N // _R,),
        in_specs=[
            pl.BlockSpec((_R, D), lambda i: (i, 0)),
            pl.BlockSpec((D, D), lambda i: (0, 0)),
            pl.BlockSpec((_R, 1), lambda i: (i, 0)),
            pl.BlockSpec((_R, 1), lambda i: (i, 0)),
        ],
        out_specs=[
            pl.BlockSpec((_R, D), lambda i: (i, 0)),
            pl.BlockSpec((_R, 1), lambda i: (i, 0)),
        ],
        out_shape=[
            jax.ShapeDtypeStruct((N, D), jnp.float32),
            jax.ShapeDtypeStruct((N, 1), jnp.float32),
        ],
    )(x, w1, dega, degb)


def _tc2(z0, z1, y1, dinv, b1, w2):
    def body(z0_ref, z1_ref, y_ref, d_ref, b_ref, w_ref, o_ref):
        t = (z0_ref[...] + z1_ref[...] + y_ref[...]) * d_ref[...] + b_ref[...]
        h = jnp.where(t >= 0, t, NEG * t)
        o_ref[...] = jnp.dot(h, w_ref[...], preferred_element_type=jnp.float32) * d_ref[...]

    return pl.pallas_call(
        body,
        grid=(N // _R,),
        in_specs=[
            pl.BlockSpec((_R, D), lambda i: (i, 0)),
            pl.BlockSpec((_R, D), lambda i: (i, 0)),
            pl.BlockSpec((_R, D), lambda i: (i, 0)),
            pl.BlockSpec((_R, 1), lambda i: (i, 0)),
            pl.BlockSpec((1, D), lambda i: (0, 0)),
            pl.BlockSpec((D, D), lambda i: (0, 0)),
        ],
        out_specs=pl.BlockSpec((_R, D), lambda i: (i, 0)),
        out_shape=jax.ShapeDtypeStruct((N, D), jnp.float32),
    )(z0, z1, y1, dinv, b1, w2)


def _tc3(z0, z1, y2, dinv, b2):
    def body(z0_ref, z1_ref, y_ref, d_ref, b_ref, o_ref):
        t = (z0_ref[...] + z1_ref[...] + y_ref[...]) * d_ref[...] + b_ref[...]
        o_ref[...] = jnp.where(t >= 0, t, NEG * t)

    return pl.pallas_call(
        body,
        grid=(N // _R,),
        in_specs=[
            pl.BlockSpec((_R, D), lambda i: (i, 0)),
            pl.BlockSpec((_R, D), lambda i: (i, 0)),
            pl.BlockSpec((_R, D), lambda i: (i, 0)),
            pl.BlockSpec((_R, 1), lambda i: (i, 0)),
            pl.BlockSpec((1, D), lambda i: (0, 0)),
        ],
        out_specs=pl.BlockSpec((_R, D), lambda i: (i, 0)),
        out_shape=jax.ShapeDtypeStruct((N, D), jnp.float32),
    )(z0, z1, y2, dinv, b2)


def kernel(x, edge_index, W1, b1, W2, b2):
    pad = EPAD - E
    sink = N + jnp.arange(pad, dtype=jnp.int32) % (NPAD - N)
    src_flat = jnp.concatenate([edge_index[0], jnp.full((pad,), N, jnp.int32)])
    dst_flat = jnp.concatenate([edge_index[1], sink])
    srcp = src_flat.reshape(EPAD // K2, K2)
    dstp = dst_flat.reshape(EPAD // K2, K2)

    deg = _deg_call(dst_flat.reshape(EPAD // K, K))
    dega = deg[0, :N].reshape(N, 1)
    degb = deg[1, :N].reshape(N, 1)

    y1, dinv = _tc1(x, W1, dega, degb)
    y1p = jnp.concatenate([y1, jnp.zeros((NPAD - N, D), jnp.float32)])
    z1 = _scat_call(y1p, srcp, dstp)

    y2 = _tc2(z1[0, :N], z1[1, :N], y1, dinv, b1.reshape(1, D), W2)
    y2p = jnp.concatenate([y2, jnp.zeros((NPAD - N, D), jnp.float32)])
    z2 = _scat_call(y2p, srcp, dstp)

    return _tc3(z2[0, :N], z2[1, :N], y2, dinv, b2.reshape(1, D))


# speedup vs baseline: 1.1410x; 1.1410x over previous
import functools

import jax
import jax.numpy as jnp
from jax import lax
from jax.experimental import pallas as pl
from jax.experimental.pallas import tpu as pltpu
from jax.experimental.pallas import tpu_sc as plsc

N = 10000
E = 320000
D = 128
NEG = 0.01

NC = 2
NS = 16
K = 128
CPT = 80
EPAD = NC * NS * CPT * K
NPAD = 10240
RPT = NPAD // NS
_mesh = plsc.VectorSubcoreMesh(core_axis_name="c", subcore_axis_name="s")


@functools.partial(
    pl.kernel,
    out_type=jax.ShapeDtypeStruct((NC, NPAD), jnp.float32),
    mesh=_mesh,
    scratch_types=[
        pltpu.VMEM_SHARED((NPAD,), jnp.float32),
        pltpu.VMEM((CPT, K), jnp.int32),
        pltpu.VMEM((K,), jnp.float32),
        pltpu.VMEM((RPT,), jnp.float32),
    ],
)
def _deg_call(dst_hbm, out_hbm, dsh, didx, ones, zbuf):
    c = lax.axis_index("c")
    s = lax.axis_index("s")
    wid = c * NS + s

    def fill_ones(i, _):
        ones[pl.ds(i * 16, 16)] = jnp.full((16,), 1.0, jnp.float32)
        return 0

    lax.fori_loop(0, K // 16, fill_ones, 0)

    def fill_z(i, _):
        zbuf[pl.ds(i * 16, 16)] = jnp.zeros((16,), jnp.float32)
        return 0

    lax.fori_loop(0, RPT // 16, fill_z, 0)

    base = s * RPT
    pltpu.sync_copy(zbuf, dsh.at[pl.ds(base, RPT)])
    plsc.subcore_barrier()

    pltpu.sync_copy(dst_hbm.at[pl.ds(wid * CPT, CPT)], didx)

    def chunk(j, _):
        pltpu.sync_copy(ones, dsh.at[didx.at[j]], add=True)
        return 0

    lax.fori_loop(0, CPT, chunk, 0)
    plsc.subcore_barrier()
    pltpu.sync_copy(dsh.at[pl.ds(base, RPT)], out_hbm.at[c, pl.ds(base, RPT)])


STG0 = 3
STG1 = 1
CLOC = 40


@functools.partial(
    pl.kernel,
    out_type=jax.ShapeDtypeStruct((NC, NPAD, D), jnp.float32),
    mesh=_mesh,
    scratch_types=[
        pltpu.VMEM_SHARED((NPAD, D), jnp.float32),
        pltpu.VMEM((CLOC, K), jnp.int32),
        pltpu.VMEM((CLOC, K), jnp.int32),
        pltpu.VMEM((K, D), jnp.float32),
        pltpu.VMEM((K, D), jnp.float32),
        pltpu.SemaphoreType.DMA,
        pltpu.SemaphoreType.DMA,
    ],
)
def _scat_call(y_hbm, src_hbm, dst_hbm, out_hbm, zsh, sidx, didx, g0, g1, sem0, sem1):
    c = lax.axis_index("c")
    s = lax.axis_index("s")
    wid = c * NS + s

    def fill_z(i, _):
        def fill_col(j, _):
            g0[i, pl.ds(j * 16, 16)] = jnp.zeros((16,), jnp.float32)
            return 0

        return lax.fori_loop(0, D // 16, fill_col, 0)

    lax.fori_loop(0, K, fill_z, 0)

    base = s * RPT
    for t in range(RPT // K):
        pltpu.sync_copy(g0, zsh.at[pl.ds(base + t * K, K)])
    plsc.subcore_barrier()

    nst = jnp.where(c == 0, STG0, STG1)
    base0 = jnp.where(c == 0, s * (STG0 * CLOC), NS * (STG0 * CLOC) + s * (STG1 * CLOC))

    def stage(st, _):
        row0 = base0 + st * CLOC
        pltpu.sync_copy(src_hbm.at[pl.ds(row0, CLOC)], sidx)
        pltpu.sync_copy(dst_hbm.at[pl.ds(row0, CLOC)], didx)

        pltpu.async_copy(y_hbm.at[sidx.at[0]], g0, sem0)

        def pair(i, _):
            a = 2 * i
            pltpu.async_copy(y_hbm.at[sidx.at[a + 1]], g1, sem1)
            pltpu.make_async_copy(y_hbm.at[sidx.at[a]], g0, sem0).wait()
            pltpu.sync_copy(g0, zsh.at[didx.at[a]], add=True)
            pltpu.async_copy(y_hbm.at[sidx.at[a + 2]], g0, sem0)
            pltpu.make_async_copy(y_hbm.at[sidx.at[a + 1]], g1, sem1).wait()
            pltpu.sync_copy(g1, zsh.at[didx.at[a + 1]], add=True)
            return 0

        lax.fori_loop(0, CLOC // 2 - 1, pair, 0)
        a = CLOC - 2
        pltpu.async_copy(y_hbm.at[sidx.at[CLOC - 1]], g1, sem1)
        pltpu.make_async_copy(y_hbm.at[sidx.at[a]], g0, sem0).wait()
        pltpu.sync_copy(g0, zsh.at[didx.at[a]], add=True)
        pltpu.make_async_copy(y_hbm.at[sidx.at[CLOC - 1]], g1, sem1).wait()
        pltpu.sync_copy(g1, zsh.at[didx.at[CLOC - 1]], add=True)
        return 0

    lax.fori_loop(0, nst, stage, 0)
    plsc.subcore_barrier()
    pltpu.sync_copy(zsh.at[pl.ds(base, RPT)], out_hbm.at[c, pl.ds(base, RPT)])


_R = 2000


def _tc1(x, w1, dega, degb):
    def body(x_ref, w_ref, da_ref, db_ref, y_ref, dinv_ref):
        d = lax.rsqrt(da_ref[...] + db_ref[...] + 1.0)
        y_ref[...] = jnp.dot(x_ref[...], w_ref[...], preferred_element_type=jnp.float32) * d
        dinv_ref[...] = d

    return pl.pallas_call(
        body,
        grid=(N // _R,),
        in_specs=[
            pl.BlockSpec((_R, D), lambda i: (i, 0)),
            pl.BlockSpec((D, D), lambda i: (0, 0)),
            pl.BlockSpec((_R, 1), lambda i: (i, 0)),
            pl.BlockSpec((_R, 1), lambda i: (i, 0)),
        ],
        out_specs=[
            pl.BlockSpec((_R, D), lambda i: (i, 0)),
            pl.BlockSpec((_R, 1), lambda i: (i, 0)),
        ],
        out_shape=[
            jax.ShapeDtypeStruct((N, D), jnp.float32),
            jax.ShapeDtypeStruct((N, 1), jnp.float32),
        ],
    )(x, w1, dega, degb)


def _tc2(z0, z1, y1, dinv, b1, w2):
    def body(z0_ref, z1_ref, y_ref, d_ref, b_ref, w_ref, o_ref):
        t = (z0_ref[...] + z1_ref[...] + y_ref[...]) * d_ref[...] + b_ref[...]
        h = jnp.where(t >= 0, t, NEG * t)
        o_ref[...] = jnp.dot(h, w_ref[...], preferred_element_type=jnp.float32) * d_ref[...]

    return pl.pallas_call(
        body,
        grid=(N // _R,),
        in_specs=[
            pl.BlockSpec((_R, D), lambda i: (i, 0)),
            pl.BlockSpec((_R, D), lambda i: (i, 0)),
            pl.BlockSpec((_R, D), lambda i: (i, 0)),
            pl.BlockSpec((_R, 1), lambda i: (i, 0)),
            pl.BlockSpec((1, D), lambda i: (0, 0)),
            pl.BlockSpec((D, D), lambda i: (0, 0)),
        ],
        out_specs=pl.BlockSpec((_R, D), lambda i: (i, 0)),
        out_shape=jax.ShapeDtypeStruct((N, D), jnp.float32),
    )(z0, z1, y1, dinv, b1, w2)


def _tc3(z0, z1, y2, dinv, b2):
    def body(z0_ref, z1_ref, y_ref, d_ref, b_ref, o_ref):
        t = (z0_ref[...] + z1_ref[...] + y_ref[...]) * d_ref[...] + b_ref[...]
        o_ref[...] = jnp.where(t >= 0, t, NEG * t)

    return pl.pallas_call(
        body,
        grid=(N // _R,),
        in_specs=[
            pl.BlockSpec((_R, D), lambda i: (i, 0)),
            pl.BlockSpec((_R, D), lambda i: (i, 0)),
            pl.BlockSpec((_R, D), lambda i: (i, 0)),
            pl.BlockSpec((_R, 1), lambda i: (i, 0)),
            pl.BlockSpec((1, D), lambda i: (0, 0)),
        ],
        out_specs=pl.BlockSpec((_R, D), lambda i: (i, 0)),
        out_shape=jax.ShapeDtypeStruct((N, D), jnp.float32),
    )(z0, z1, y2, dinv, b2)


def kernel(x, edge_index, W1, b1, W2, b2):
    pad = EPAD - E
    sink = N + jnp.arange(pad, dtype=jnp.int32) % (NPAD - N)
    srcp = jnp.concatenate([edge_index[0], jnp.full((pad,), N, jnp.int32)]).reshape(EPAD // K, K)
    dstp = jnp.concatenate([edge_index[1], sink]).reshape(EPAD // K, K)

    deg = _deg_call(dstp)
    dega = deg[0, :N].reshape(N, 1)
    degb = deg[1, :N].reshape(N, 1)

    y1, dinv = _tc1(x, W1, dega, degb)
    y1p = jnp.concatenate([y1, jnp.zeros((NPAD - N, D), jnp.float32)])
    z1 = _scat_call(y1p, srcp, dstp)

    y2 = _tc2(z1[0, :N], z1[1, :N], y1, dinv, b1.reshape(1, D), W2)
    y2p = jnp.concatenate([y2, jnp.zeros((NPAD - N, D), jnp.float32)])
    z2 = _scat_call(y2p, srcp, dstp)

    return _tc3(z2[0, :N], z2[1, :N], y2, dinv, b2.reshape(1, D))
